# baseline (speedup 1.0000x reference)
import math

import jax
import jax.numpy as jnp
from jax.experimental import pallas as pl
from jax.experimental.pallas import tpu as pltpu

B = 4096
KNOW = 4096
D_MODEL = 1024
TOP_K = 64
INV_SQRT_D = 1.0 / math.sqrt(D_MODEL)

ROW_BLK = 128


def _proj_kernel(ae_ref, pe_ref, w_ref, out_ref):
    r = 0.5 * (ae_ref[...] + pe_ref[...])
    out_ref[...] = (
        jax.lax.dot_general(
            r, w_ref[...], (((1,), (1,)), ((), ())),
            preferred_element_type=jnp.float32,
        )
        + r
    )


def _f32_sort_key(x):
    u = jax.lax.bitcast_convert_type(x, jnp.uint32)
    neg = (u >> 31) == 1
    return jnp.where(neg, ~u, u | jnp.uint32(0x80000000))


def _key_to_f32(k):
    neg = (k >> 31) == 0
    u = jnp.where(neg, ~k, k & jnp.uint32(0x7FFFFFFF))
    return jax.lax.bitcast_convert_type(u, jnp.float32)


def _mega_kernel(aeq_ref, peq_ref, wq_ref, pkv_ref, key_ref, out_ref):
    sims = (
        jax.lax.dot_general(
            peq_ref[...], pkv_ref[...], (((1,), (1,)), ((), ())),
            preferred_element_type=jnp.float32,
        )
        * INV_SQRT_D
    )
    r = 0.5 * (aeq_ref[...] + peq_ref[...])
    query = (
        jax.lax.dot_general(
            r, wq_ref[...], (((1,), (1,)), ((), ())),
            preferred_element_type=jnp.float32,
        )
        + r
    )
    att = (
        jax.lax.dot_general(
            query, key_ref[...], (((1,), (1,)), ((), ())),
            preferred_element_type=jnp.float32,
        )
        * INV_SQRT_D
    )
    sims_bf = sims.astype(jnp.bfloat16)
    prefix = jnp.zeros((ROW_BLK, 1), dtype=jnp.uint32)

    def u16key_to_bf16(t):
        neg = (t >> 15) == 0
        u = jnp.where(neg, ~t & jnp.uint32(0xFFFF), t & jnp.uint32(0x7FFF))
        return jax.lax.bitcast_convert_type(u << 16, jnp.float32).astype(jnp.bfloat16)

    def count_cmp(mask):
        m = mask.astype(jnp.bfloat16)
        part = jnp.sum(m.reshape(ROW_BLK, 32, 128), axis=1)
        return jnp.sum(part.astype(jnp.float32), axis=1, keepdims=True)

    for b in range(15, -1, -1):
        t = prefix | jnp.uint32(1 << b)
        cnt = count_cmp(sims_bf >= u16key_to_bf16(t))
        prefix = jnp.where(cnt >= TOP_K, t, prefix)
    t_bf = u16key_to_bf16(prefix)

    n_gt = count_cmp(sims_bf > t_bf)
    rprime = (jnp.float32(TOP_K) - n_gt).astype(jnp.int32)
    tievals = jnp.where(sims_bf == t_bf, sims, jnp.float32(-jnp.inf))

    def _cond(c):
        r, _, _ = c
        return jnp.any(r > 0)

    def _body(c):
        r, vals, kth = c
        m = jnp.max(vals, axis=1, keepdims=True)
        c_eq = jnp.sum((vals == m).astype(jnp.int32), axis=1, keepdims=True)
        active = r > 0
        done = active & (r <= c_eq)
        kth = jnp.where(done, m, kth)
        r = jnp.where(active & ~done, r - c_eq, 0)
        vals = jnp.where(vals == m, jnp.float32(-jnp.inf), vals)
        return r, vals, kth

    _, _, kth = jax.lax.while_loop(
        _cond, _body,
        (rprime, tievals, jnp.zeros((ROW_BLK, 1), jnp.float32)),
    )
    out_ref[...] = jnp.where(sims < kth, jnp.float32(0.0), att)


def kernel(ae_q, ae_kv, pe_q, pe_kv, Wq, Wk):
    n_row = B // ROW_BLK

    key_mat = pl.pallas_call(
        _proj_kernel,
        grid=(n_row,),
        in_specs=[
            pl.BlockSpec((ROW_BLK, D_MODEL), lambda i: (i, 0)),
            pl.BlockSpec((ROW_BLK, D_MODEL), lambda i: (i, 0)),
            pl.BlockSpec((D_MODEL, D_MODEL), lambda i: (0, 0)),
        ],
        out_specs=pl.BlockSpec((ROW_BLK, D_MODEL), lambda i: (i, 0)),
        out_shape=jax.ShapeDtypeStruct((KNOW, D_MODEL), jnp.float32),
        compiler_params=pltpu.CompilerParams(
            dimension_semantics=("arbitrary",),
        ),
    )(ae_kv, pe_kv, Wk)

    out = pl.pallas_call(
        _mega_kernel,
        grid=(n_row,),
        in_specs=[
            pl.BlockSpec((ROW_BLK, D_MODEL), lambda i: (i, 0)),
            pl.BlockSpec((ROW_BLK, D_MODEL), lambda i: (i, 0)),
            pl.BlockSpec((D_MODEL, D_MODEL), lambda i: (0, 0)),
            pl.BlockSpec((KNOW, D_MODEL), lambda i: (0, 0)),
            pl.BlockSpec((KNOW, D_MODEL), lambda i: (0, 0)),
        ],
        out_specs=pl.BlockSpec((ROW_BLK, KNOW), lambda i: (i, 0)),
        out_shape=jax.ShapeDtypeStruct((B, KNOW), jnp.float32),
        compiler_params=pltpu.CompilerParams(
            dimension_semantics=("arbitrary",),
        ),
    )(ae_q, pe_q, Wq, pe_kv, key_mat)
    return out


# speedup vs baseline: 1.0773x; 1.0773x over previous
import math

import jax
import jax.numpy as jnp
from jax.experimental import pallas as pl
from jax.experimental.pallas import tpu as pltpu

B = 4096
KNOW = 4096
D_MODEL = 1024
TOP_K = 64
INV_SQRT_D = 1.0 / math.sqrt(D_MODEL)

ROW_BLK = 128


def _proj_kernel(ae_ref, pe_ref, w_ref, out_ref):
    r = 0.5 * (ae_ref[...] + pe_ref[...])
    out_ref[...] = (
        jax.lax.dot_general(
            r, w_ref[...], (((1,), (1,)), ((), ())),
            preferred_element_type=jnp.float32,
        )
        + r
    )


def _f32_sort_key(x):
    u = jax.lax.bitcast_convert_type(x, jnp.uint32)
    neg = (u >> 31) == 1
    return jnp.where(neg, ~u, u | jnp.uint32(0x80000000))


def _key_to_f32(k):
    neg = (k >> 31) == 0
    u = jnp.where(neg, ~k, k & jnp.uint32(0x7FFFFFFF))
    return jax.lax.bitcast_convert_type(u, jnp.float32)


def _mega_kernel(aeq_ref, peq_ref, wq_ref, pkv_ref, key_ref, out_ref):
    sims = (
        jax.lax.dot_general(
            peq_ref[...], pkv_ref[...], (((1,), (1,)), ((), ())),
            preferred_element_type=jnp.float32,
        )
        * INV_SQRT_D
    )
    r = 0.5 * (aeq_ref[...] + peq_ref[...])
    query = (
        jax.lax.dot_general(
            r, wq_ref[...], (((1,), (1,)), ((), ())),
            preferred_element_type=jnp.float32,
        )
        + r
    )
    att = (
        jax.lax.dot_general(
            query, key_ref[...], (((1,), (1,)), ((), ())),
            preferred_element_type=jnp.float32,
        )
        * INV_SQRT_D
    )
    prefix = jnp.zeros((ROW_BLK, 1), dtype=jnp.uint32)
    for b in range(31, -1, -1):
        t_f = _key_to_f32(prefix | jnp.uint32(1 << b))
        m = jnp.where(sims >= t_f, jnp.float32(1.0), jnp.float32(0.0))
        part = jnp.sum(m.reshape(ROW_BLK, 32, 128), axis=1)
        cnt = jnp.sum(part, axis=1, keepdims=True)
        prefix = jnp.where(cnt >= jnp.float32(TOP_K),
                           prefix | jnp.uint32(1 << b), prefix)
    kth = _key_to_f32(prefix)
    out_ref[...] = jnp.where(sims < kth, jnp.float32(0.0), att)


def kernel(ae_q, ae_kv, pe_q, pe_kv, Wq, Wk):
    n_row = B // ROW_BLK

    key_mat = pl.pallas_call(
        _proj_kernel,
        grid=(n_row,),
        in_specs=[
            pl.BlockSpec((ROW_BLK, D_MODEL), lambda i: (i, 0)),
            pl.BlockSpec((ROW_BLK, D_MODEL), lambda i: (i, 0)),
            pl.BlockSpec((D_MODEL, D_MODEL), lambda i: (0, 0)),
        ],
        out_specs=pl.BlockSpec((ROW_BLK, D_MODEL), lambda i: (i, 0)),
        out_shape=jax.ShapeDtypeStruct((KNOW, D_MODEL), jnp.float32),
        compiler_params=pltpu.CompilerParams(
            dimension_semantics=("arbitrary",),
        ),
    )(ae_kv, pe_kv, Wk)

    out = pl.pallas_call(
        _mega_kernel,
        grid=(n_row,),
        in_specs=[
            pl.BlockSpec((ROW_BLK, D_MODEL), lambda i: (i, 0)),
            pl.BlockSpec((ROW_BLK, D_MODEL), lambda i: (i, 0)),
            pl.BlockSpec((D_MODEL, D_MODEL), lambda i: (0, 0)),
            pl.BlockSpec((KNOW, D_MODEL), lambda i: (0, 0)),
            pl.BlockSpec((KNOW, D_MODEL), lambda i: (0, 0)),
        ],
        out_specs=pl.BlockSpec((ROW_BLK, KNOW), lambda i: (i, 0)),
        out_shape=jax.ShapeDtypeStruct((B, KNOW), jnp.float32),
        compiler_params=pltpu.CompilerParams(
            dimension_semantics=("arbitrary",),
        ),
    )(ae_q, pe_q, Wq, pe_kv, key_mat)
    return out


# speedup vs baseline: 2.1194x; 1.9673x over previous
import math

import jax
import jax.numpy as jnp
from jax.experimental import pallas as pl
from jax.experimental.pallas import tpu as pltpu

B = 4096
KNOW = 4096
D_MODEL = 1024
TOP_K = 64
INV_SQRT_D = 1.0 / math.sqrt(D_MODEL)

ROW_BLK = 128


def _proj_kernel(ae_ref, pe_ref, w_ref, out_ref):
    r = 0.5 * (ae_ref[...] + pe_ref[...])
    out_ref[...] = (
        jax.lax.dot_general(
            r, w_ref[...], (((1,), (1,)), ((), ())),
            preferred_element_type=jnp.float32,
        )
        + r
    )


def _f32_sort_key(x):
    u = jax.lax.bitcast_convert_type(x, jnp.uint32)
    neg = (u >> 31) == 1
    return jnp.where(neg, ~u, u | jnp.uint32(0x80000000))


def _key_to_f32(k):
    neg = (k >> 31) == 0
    u = jnp.where(neg, ~k, k & jnp.uint32(0x7FFFFFFF))
    return jax.lax.bitcast_convert_type(u, jnp.float32)


def _mega_kernel(aeq_ref, peq_ref, wq_ref, pkv_ref, key_ref, out_ref):
    sims = (
        jax.lax.dot_general(
            peq_ref[...], pkv_ref[...], (((1,), (1,)), ((), ())),
            preferred_element_type=jnp.float32,
        )
        * INV_SQRT_D
    )
    r = 0.5 * (aeq_ref[...] + peq_ref[...])
    query = (
        jax.lax.dot_general(
            r, wq_ref[...], (((1,), (1,)), ((), ())),
            preferred_element_type=jnp.float32,
        )
        + r
    )
    att = (
        jax.lax.dot_general(
            query, key_ref[...], (((1,), (1,)), ((), ())),
            preferred_element_type=jnp.float32,
        )
        * INV_SQRT_D
    )
    prefix = jnp.zeros((ROW_BLK, 1), dtype=jnp.uint32)
    for b in range(31, -1, -1):
        t_f = _key_to_f32(prefix | jnp.uint32(1 << b))
        cnt = jnp.sum((sims >= t_f).astype(jnp.int32), axis=1, keepdims=True)
        prefix = jnp.where(cnt >= TOP_K, prefix | jnp.uint32(1 << b), prefix)
    kth = _key_to_f32(prefix)
    out_ref[...] = jnp.where(sims < kth, jnp.float32(0.0), att)


def kernel(ae_q, ae_kv, pe_q, pe_kv, Wq, Wk):
    n_row = B // ROW_BLK

    key_mat = pl.pallas_call(
        _proj_kernel,
        grid=(n_row,),
        in_specs=[
            pl.BlockSpec((ROW_BLK, D_MODEL), lambda i: (i, 0)),
            pl.BlockSpec((ROW_BLK, D_MODEL), lambda i: (i, 0)),
            pl.BlockSpec((D_MODEL, D_MODEL), lambda i: (0, 0)),
        ],
        out_specs=pl.BlockSpec((ROW_BLK, D_MODEL), lambda i: (i, 0)),
        out_shape=jax.ShapeDtypeStruct((KNOW, D_MODEL), jnp.float32),
        compiler_params=pltpu.CompilerParams(
            dimension_semantics=("arbitrary",),
        ),
    )(ae_kv, pe_kv, Wk)

    out = pl.pallas_call(
        _mega_kernel,
        grid=(n_row,),
        in_specs=[
            pl.BlockSpec((ROW_BLK, D_MODEL), lambda i: (i, 0)),
            pl.BlockSpec((ROW_BLK, D_MODEL), lambda i: (i, 0)),
            pl.BlockSpec((D_MODEL, D_MODEL), lambda i: (0, 0)),
            pl.BlockSpec((KNOW, D_MODEL), lambda i: (0, 0)),
            pl.BlockSpec((KNOW, D_MODEL), lambda i: (0, 0)),
        ],
        out_specs=pl.BlockSpec((ROW_BLK, KNOW), lambda i: (i, 0)),
        out_shape=jax.ShapeDtypeStruct((B, KNOW), jnp.float32),
        compiler_params=pltpu.CompilerParams(
            dimension_semantics=("arbitrary",),
        ),
    )(ae_q, pe_q, Wq, pe_kv, key_mat)
    return out


# speedup vs baseline: 2.6398x; 1.2455x over previous
import math

import jax
import jax.numpy as jnp
from jax.experimental import pallas as pl
from jax.experimental.pallas import tpu as pltpu

B = 4096
KNOW = 4096
D_MODEL = 1024
TOP_K = 64
INV_SQRT_D = 1.0 / math.sqrt(D_MODEL)

ROW_BLK = 256


def _proj_kernel(ae_ref, pe_ref, w_ref, out_ref):
    r = 0.5 * (ae_ref[...] + pe_ref[...])
    out_ref[...] = (
        jax.lax.dot_general(
            r, w_ref[...], (((1,), (1,)), ((), ())),
            preferred_element_type=jnp.float32,
        )
        + r
    )


def _f32_sort_key(x):
    u = jax.lax.bitcast_convert_type(x, jnp.uint32)
    neg = (u >> 31) == 1
    return jnp.where(neg, ~u, u | jnp.uint32(0x80000000))


def _key_to_f32(k):
    neg = (k >> 31) == 0
    u = jnp.where(neg, ~k, k & jnp.uint32(0x7FFFFFFF))
    return jax.lax.bitcast_convert_type(u, jnp.float32)


def _mega_kernel(aeq_ref, peq_ref, wq_ref, pkv_ref, key_ref, out_ref):
    sims = (
        jax.lax.dot_general(
            peq_ref[...], pkv_ref[...], (((1,), (1,)), ((), ())),
            preferred_element_type=jnp.float32,
        )
        * INV_SQRT_D
    )
    r = 0.5 * (aeq_ref[...] + peq_ref[...])
    query = (
        jax.lax.dot_general(
            r, wq_ref[...], (((1,), (1,)), ((), ())),
            preferred_element_type=jnp.float32,
        )
        + r
    )
    att = (
        jax.lax.dot_general(
            query, key_ref[...], (((1,), (1,)), ((), ())),
            preferred_element_type=jnp.float32,
        )
        * INV_SQRT_D
    )
    prefix = jnp.zeros((ROW_BLK, 1), dtype=jnp.uint32)
    for b in range(31, -1, -1):
        t_f = _key_to_f32(prefix | jnp.uint32(1 << b))
        cnt = jnp.sum((sims >= t_f).astype(jnp.int32), axis=1, keepdims=True)
        prefix = jnp.where(cnt >= TOP_K, prefix | jnp.uint32(1 << b), prefix)
    kth = _key_to_f32(prefix)
    out_ref[...] = jnp.where(sims < kth, jnp.float32(0.0), att)


def kernel(ae_q, ae_kv, pe_q, pe_kv, Wq, Wk):
    n_row = B // ROW_BLK

    key_mat = pl.pallas_call(
        _proj_kernel,
        grid=(n_row,),
        in_specs=[
            pl.BlockSpec((ROW_BLK, D_MODEL), lambda i: (i, 0)),
            pl.BlockSpec((ROW_BLK, D_MODEL), lambda i: (i, 0)),
            pl.BlockSpec((D_MODEL, D_MODEL), lambda i: (0, 0)),
        ],
        out_specs=pl.BlockSpec((ROW_BLK, D_MODEL), lambda i: (i, 0)),
        out_shape=jax.ShapeDtypeStruct((KNOW, D_MODEL), jnp.float32),
        compiler_params=pltpu.CompilerParams(
            dimension_semantics=("arbitrary",),
        ),
    )(ae_kv, pe_kv, Wk)

    out = pl.pallas_call(
        _mega_kernel,
        grid=(n_row,),
        in_specs=[
            pl.BlockSpec((ROW_BLK, D_MODEL), lambda i: (i, 0)),
            pl.BlockSpec((ROW_BLK, D_MODEL), lambda i: (i, 0)),
            pl.BlockSpec((D_MODEL, D_MODEL), lambda i: (0, 0)),
            pl.BlockSpec((KNOW, D_MODEL), lambda i: (0, 0)),
            pl.BlockSpec((KNOW, D_MODEL), lambda i: (0, 0)),
        ],
        out_specs=pl.BlockSpec((ROW_BLK, KNOW), lambda i: (i, 0)),
        out_shape=jax.ShapeDtypeStruct((B, KNOW), jnp.float32),
        compiler_params=pltpu.CompilerParams(
            dimension_semantics=("arbitrary",),
            vmem_limit_bytes=112 * 1024 * 1024,
        ),
    )(ae_q, pe_q, Wq, pe_kv, key_mat)
    return out


# speedup vs baseline: 2.6403x; 1.0002x over previous
import math

import jax
import jax.numpy as jnp
from jax.experimental import pallas as pl
from jax.experimental.pallas import tpu as pltpu

B = 4096
KNOW = 4096
D_MODEL = 1024
TOP_K = 64
INV_SQRT_D = 1.0 / math.sqrt(D_MODEL)

ROW_BLK = 256


def _proj_kernel(ae_ref, pe_ref, w_ref, out_ref):
    r = 0.5 * (ae_ref[...] + pe_ref[...])
    out_ref[...] = (
        jax.lax.dot_general(
            r, w_ref[...], (((1,), (1,)), ((), ())),
            preferred_element_type=jnp.float32,
        )
        + r
    )


def _f32_sort_key(x):
    u = jax.lax.bitcast_convert_type(x, jnp.uint32)
    neg = (u >> 31) == 1
    return jnp.where(neg, ~u, u | jnp.uint32(0x80000000))


def _key_to_f32(k):
    neg = (k >> 31) == 0
    u = jnp.where(neg, ~k, k & jnp.uint32(0x7FFFFFFF))
    return jax.lax.bitcast_convert_type(u, jnp.float32)


def _mega_kernel(aeq_ref, peq_ref, wq_ref, pkv_ref, key_ref, out_ref):
    sims = (
        jax.lax.dot_general(
            peq_ref[...], pkv_ref[...], (((1,), (1,)), ((), ())),
            preferred_element_type=jnp.float32,
        )
        * INV_SQRT_D
    )
    r = 0.5 * (aeq_ref[...] + peq_ref[...])
    query = (
        jax.lax.dot_general(
            r, wq_ref[...], (((1,), (1,)), ((), ())),
            preferred_element_type=jnp.float32,
        )
        + r
    )
    att = (
        jax.lax.dot_general(
            query, key_ref[...], (((1,), (1,)), ((), ())),
            preferred_element_type=jnp.float32,
        )
        * INV_SQRT_D
    )
    skey = _f32_sort_key(sims)
    prefix = jnp.zeros((ROW_BLK, 1), dtype=jnp.uint32)
    for b in range(31, -1, -1):
        t = prefix | jnp.uint32(1 << b)
        cnt = jnp.sum((skey >= t).astype(jnp.int32), axis=1, keepdims=True)
        prefix = jnp.where(cnt >= TOP_K, t, prefix)
    kth = _key_to_f32(prefix)
    out_ref[...] = jnp.where(sims < kth, jnp.float32(0.0), att)


def kernel(ae_q, ae_kv, pe_q, pe_kv, Wq, Wk):
    n_row = B // ROW_BLK

    key_mat = pl.pallas_call(
        _proj_kernel,
        grid=(n_row,),
        in_specs=[
            pl.BlockSpec((ROW_BLK, D_MODEL), lambda i: (i, 0)),
            pl.BlockSpec((ROW_BLK, D_MODEL), lambda i: (i, 0)),
            pl.BlockSpec((D_MODEL, D_MODEL), lambda i: (0, 0)),
        ],
        out_specs=pl.BlockSpec((ROW_BLK, D_MODEL), lambda i: (i, 0)),
        out_shape=jax.ShapeDtypeStruct((KNOW, D_MODEL), jnp.float32),
        compiler_params=pltpu.CompilerParams(
            dimension_semantics=("arbitrary",),
        ),
    )(ae_kv, pe_kv, Wk)

    out = pl.pallas_call(
        _mega_kernel,
        grid=(n_row,),
        in_specs=[
            pl.BlockSpec((ROW_BLK, D_MODEL), lambda i: (i, 0)),
            pl.BlockSpec((ROW_BLK, D_MODEL), lambda i: (i, 0)),
            pl.BlockSpec((D_MODEL, D_MODEL), lambda i: (0, 0)),
            pl.BlockSpec((KNOW, D_MODEL), lambda i: (0, 0)),
            pl.BlockSpec((KNOW, D_MODEL), lambda i: (0, 0)),
        ],
        out_specs=pl.BlockSpec((ROW_BLK, KNOW), lambda i: (i, 0)),
        out_shape=jax.ShapeDtypeStruct((B, KNOW), jnp.float32),
        compiler_params=pltpu.CompilerParams(
            dimension_semantics=("arbitrary",),
            vmem_limit_bytes=112 * 1024 * 1024,
        ),
    )(ae_q, pe_q, Wq, pe_kv, key_mat)
    return out
